# baseline (speedup 1.0000x reference)
import functools

import jax
import jax.numpy as jnp
from jax import lax
from jax.experimental import pallas as pl
from jax.experimental.pallas import tpu as pltpu
from jax.experimental.pallas import tpu_sc as plsc

H = 128
N = 10000
E = 320000
G = 64
OUT = 5

NC = 2
NS = 16
L = 16
NW = NC * NS
EPW = E // NW
CK = 80
NCHUNK = EPW // CK
NBUF = 4
NSUP = NCHUNK // NBUF
NTAIL = NCHUNK - NSUP * NBUF
SCAT_BYTES = CK * H * 4
RPT = 624
TAIL = N - NS * RPT

_NV = H // L

_GD = lax.GatherDimensionNumbers(
    offset_dims=(), collapsed_slice_dims=(0,), start_index_map=(0,))


def _bcast_lane(vec, j):
    idx = jnp.full((L, 1), j, jnp.int32)
    return lax.gather(vec, idx, _GD, (1,),
                      mode=lax.GatherScatterMode.PROMISE_IN_BOUNDS)


def _zero_acc(s, zero_h, acc):
    pltpu.sync_copy(zero_h.at[pl.ds(0, RPT)], acc.at[pl.ds(s * RPT, RPT)])

    @pl.when(s == 0)
    def _zero_tail():
        pltpu.sync_copy(zero_h.at[pl.ds(0, TAIL)], acc.at[pl.ds(NS * RPT, TAIL)])


def _write_out(c, s, acc, out_h):
    pltpu.sync_copy(acc.at[pl.ds(s * RPT, RPT)],
                    out_h.at[c, pl.ds(s * RPT, RPT)])

    @pl.when(s == 0)
    def _write_tail():
        pltpu.sync_copy(acc.at[pl.ds(NS * RPT, TAIL)],
                        out_h.at[c, pl.ds(NS * RPT, TAIL)])


def _sc_layer1(pk_h, ew_h, nw_h, coef_h, zero_h, out_h, *scr):
    nwg = scr[0:NBUF]
    mbuf = scr[NBUF:2 * NBUF]
    pidx, ewvb, coefv, acc, semP, semG, semS = scr[2 * NBUF:]
    c = lax.axis_index("c")
    s = lax.axis_index("s")
    wid = c * NS + s
    pltpu.sync_copy(coef_h, coefv)
    _zero_acc(s, zero_h, acc)
    plsc.subcore_barrier()
    w0 = [coefv[0, pl.ds(L * v, L)] for v in range(_NV)]
    we = [coefv[1, pl.ds(L * v, L)] for v in range(_NV)]
    cc = [coefv[2, pl.ds(L * v, L)] for v in range(_NV)]

    def chunk_compute(b):
        def group(q, cr):
            b16 = q * L
            nw16 = nwg[b][pl.ds(b16, L)]
            ew16 = ewvb[b, 0, pl.ds(b16, L)]

            def edge(jj, cj):
                j0 = jj * 2
                nwb0 = _bcast_lane(nw16, j0)
                ewb0 = _bcast_lane(ew16, j0)
                nwb1 = _bcast_lane(nw16, j0 + 1)
                ewb1 = _bcast_lane(ew16, j0 + 1)
                for v in range(_NV):
                    t0 = nwb0 * w0[v] + (ewb0 * we[v] + cc[v])
                    t1 = nwb1 * w0[v] + (ewb1 * we[v] + cc[v])
                    mbuf[b][b16 + j0, pl.ds(L * v, L)] = jnp.maximum(t0, 0.0)
                    mbuf[b][b16 + j0 + 1, pl.ds(L * v, L)] = jnp.maximum(t1, 0.0)
                return cj

            lax.fori_loop(0, L // 2, edge, 0)
            return cr

        lax.fori_loop(0, CK // L, group, 0)

    def superiter(g, carry):
        @pl.when(g > 0)
        def _drain_prev():
            for b in range(NBUF):
                pltpu.make_async_copy(mbuf[b], acc.at[pidx.at[b, 1]],
                                      semS.at[b]).wait()

        gc0 = wid * NCHUNK + g * NBUF
        hp = pltpu.async_copy(pk_h.at[pl.ds(gc0, NBUF)], pidx, semP)
        he = pltpu.async_copy(ew_h.at[pl.ds(gc0, NBUF)], ewvb, semP)
        hp.wait()
        he.wait()
        gh = [pltpu.async_copy(nw_h.at[pidx.at[b, 0]], nwg[b], semG.at[b])
              for b in range(NBUF)]
        for b in range(NBUF):
            gh[b].wait()
            chunk_compute(b)
            pltpu.async_copy(mbuf[b], acc.at[pidx.at[b, 1]], semS.at[b],
                             add=True)
        return carry

    lax.fori_loop(0, NSUP, superiter, 0)
    for b in range(NBUF):
        pltpu.make_async_copy(mbuf[b], acc.at[pidx.at[b, 1]], semS.at[b]).wait()
    for t in range(NTAIL):
        gc = wid * NCHUNK + NSUP * NBUF + t
        pltpu.sync_copy(pk_h.at[pl.ds(gc, 1)], pidx.at[pl.ds(0, 1)])
        pltpu.sync_copy(ew_h.at[pl.ds(gc, 1)], ewvb.at[pl.ds(0, 1)])
        pltpu.async_copy(nw_h.at[pidx.at[0, 0]], nwg[0], semG.at[0]).wait()
        chunk_compute(0)
        pltpu.sync_copy(mbuf[0], acc.at[pidx.at[0, 1]], add=True)
    plsc.subcore_barrier()
    _write_out(c, s, acc, out_h)


def _sc_layer2(pk_h, ew_h, x_h, coef_h, zero_h, out_h, *scr):
    rows = scr[0:NBUF]
    pidx, ewvb, coefv, acc, semP, semG, semS = scr[NBUF:]
    c = lax.axis_index("c")
    s = lax.axis_index("s")
    wid = c * NS + s
    pltpu.sync_copy(coef_h, coefv)
    _zero_acc(s, zero_h, acc)
    plsc.subcore_barrier()
    we = [coefv[0, pl.ds(L * v, L)] for v in range(_NV)]
    eb = [coefv[1, pl.ds(L * v, L)] for v in range(_NV)]

    def chunk_compute(b):
        def group(q, cr):
            b16 = q * L
            ew16 = ewvb[b, 0, pl.ds(b16, L)]

            def edge(jj, cj):
                j0 = jj * 2
                ewb0 = _bcast_lane(ew16, j0)
                ewb1 = _bcast_lane(ew16, j0 + 1)
                for v in range(_NV):
                    sl = pl.ds(L * v, L)
                    t0 = rows[b][b16 + j0, sl] + (ewb0 * we[v] + eb[v])
                    t1 = rows[b][b16 + j0 + 1, sl] + (ewb1 * we[v] + eb[v])
                    rows[b][b16 + j0, sl] = jnp.maximum(t0, 0.0)
                    rows[b][b16 + j0 + 1, sl] = jnp.maximum(t1, 0.0)
                return cj

            lax.fori_loop(0, L // 2, edge, 0)
            return cr

        lax.fori_loop(0, CK // L, group, 0)

    def superiter(g, carry):
        @pl.when(g > 0)
        def _drain_prev():
            for b in range(NBUF):
                pltpu.make_async_copy(rows[b], acc.at[pidx.at[b, 1]],
                                      semS.at[b]).wait()

        gc0 = wid * NCHUNK + g * NBUF
        hp = pltpu.async_copy(pk_h.at[pl.ds(gc0, NBUF)], pidx, semP)
        he = pltpu.async_copy(ew_h.at[pl.ds(gc0, NBUF)], ewvb, semP)
        hp.wait()
        he.wait()
        gh = [pltpu.async_copy(x_h.at[pidx.at[b, 0]], rows[b], semG.at[b])
              for b in range(NBUF)]
        for b in range(NBUF):
            gh[b].wait()
            chunk_compute(b)
            pltpu.async_copy(rows[b], acc.at[pidx.at[b, 1]], semS.at[b],
                             add=True)
        return carry

    lax.fori_loop(0, NSUP, superiter, 0)
    for b in range(NBUF):
        pltpu.make_async_copy(rows[b], acc.at[pidx.at[b, 1]], semS.at[b]).wait()
    for t in range(NTAIL):
        gc = wid * NCHUNK + NSUP * NBUF + t
        pltpu.sync_copy(pk_h.at[pl.ds(gc, 1)], pidx.at[pl.ds(0, 1)])
        pltpu.sync_copy(ew_h.at[pl.ds(gc, 1)], ewvb.at[pl.ds(0, 1)])
        pltpu.async_copy(x_h.at[pidx.at[0, 0]], rows[0], semG.at[0]).wait()
        chunk_compute(0)
        pltpu.sync_copy(rows[0], acc.at[pidx.at[0, 1]], add=True)
    plsc.subcore_barrier()
    _write_out(c, s, acc, out_h)


@functools.cache
def _build_sc_kernels():
    mesh = plsc.VectorSubcoreMesh(
        core_axis_name="c", subcore_axis_name="s",
        num_cores=NC, num_subcores=NS)
    def ring_scratch(n_row_rings, coef_rows):
        return (
            [pltpu.VMEM((CK,), jnp.float32)
             for _ in range((n_row_rings - 1) * NBUF)]
            + [pltpu.VMEM((CK, H), jnp.float32) for _ in range(NBUF)]
            + [
                pltpu.VMEM((NBUF, 2, CK), jnp.int32),
                pltpu.VMEM((NBUF, 1, CK), jnp.float32),
                pltpu.VMEM((coef_rows, H), jnp.float32),
                pltpu.VMEM_SHARED((N, H), jnp.float32),
                pltpu.SemaphoreType.DMA,
                pltpu.SemaphoreType.DMA((NBUF,)),
                pltpu.SemaphoreType.DMA((NBUF,)),
            ]
        )

    sc_l1 = pl.kernel(
        _sc_layer1,
        out_type=jax.ShapeDtypeStruct((NC, N, H), jnp.float32),
        mesh=mesh,
        scratch_types=ring_scratch(2, 3),
    )
    sc_l2 = pl.kernel(
        _sc_layer2,
        out_type=jax.ShapeDtypeStruct((NC, N, H), jnp.float32),
        mesh=mesh,
        scratch_types=ring_scratch(1, 2),
    )
    return sc_l1, sc_l2


def _bn_rows(t, g, be):
    m = jnp.mean(t, axis=0, keepdims=True)
    v = jnp.mean((t - m) ** 2, axis=0, keepdims=True)
    return g * (t - m) * lax.rsqrt(v + 1e-5) + be


def _tc_dense1(nw_ref, agg_ref, w0_ref, b0_ref, w1_ref, b1_ref,
               g1_ref, be1_ref, w2_ref, b2_ref, out_ref):
    x = nw_ref[...] * w0_ref[...] + b0_ref[...]
    h = x + agg_ref[0] + agg_ref[1]
    t = jnp.dot(h, w1_ref[...], preferred_element_type=jnp.float32) + b1_ref[...]
    t = jnp.maximum(t, 0.0)
    tn = _bn_rows(t, g1_ref[...], be1_ref[...])
    y = jnp.dot(tn, w2_ref[...], preferred_element_type=jnp.float32) + b2_ref[...]
    out_ref[...] = jnp.maximum(y, 0.0)


def _tc_dense2(x_ref, agg_ref, batch_ref, w1_ref, b1_ref, g1_ref, be1_ref,
               w2_ref, b2_ref, mw1_ref, mb1_ref, mg_ref, mbe_ref,
               mw2_ref, mb2_ref, out_ref):
    h = x_ref[...] + agg_ref[0] + agg_ref[1]
    t = jnp.dot(h, w1_ref[...], preferred_element_type=jnp.float32) + b1_ref[...]
    t = jnp.maximum(t, 0.0)
    tn = _bn_rows(t, g1_ref[...], be1_ref[...])
    y = jnp.dot(tn, w2_ref[...], preferred_element_type=jnp.float32) + b2_ref[...]
    x3 = jnp.maximum(y, 0.0)
    oh = (batch_ref[...] == lax.broadcasted_iota(jnp.int32, (1, G), 1))
    pooled = lax.dot_general(oh.astype(jnp.float32), x3,
                             (((0,), (0,)), ((), ())),
                             preferred_element_type=jnp.float32)
    hh = jnp.dot(pooled, mw1_ref[...], preferred_element_type=jnp.float32)
    hh = jnp.maximum(hh + mb1_ref[...], 0.0)
    hn = _bn_rows(hh, mg_ref[...], mbe_ref[...])
    logits = jnp.dot(hn, mw2_ref[...], preferred_element_type=jnp.float32)
    logits = logits + mb2_ref[...]
    z = logits - jnp.max(logits, axis=1, keepdims=True)
    out_ref[...] = z - jnp.log(jnp.sum(jnp.exp(z), axis=1, keepdims=True))


_tc1 = pl.pallas_call(
    _tc_dense1, out_shape=jax.ShapeDtypeStruct((N, H), jnp.float32))
_tc2 = pl.pallas_call(
    _tc_dense2, out_shape=jax.ShapeDtypeStruct((G, OUT), jnp.float32))


def kernel(edge_index, edge_weight, node_weight, batch,
           node_W, node_b, edge_W, edge_b,
           c1_W1, c1_b1, c1_g, c1_be, c1_W2, c1_b2,
           c2_W1, c2_b1, c2_g, c2_be, c2_W2, c2_b2,
           m_W1, m_b1, m_g, m_be, m_W2, m_b2):
    sc_l1, sc_l2 = _build_sc_kernels()
    src = edge_index[0]
    dst = edge_index[1]
    pk = jnp.stack([src.reshape(-1, CK), dst.reshape(-1, CK)], axis=1)
    ewk = edge_weight.reshape(-1, 1, CK)
    coef1 = jnp.stack([node_W[0], edge_W[0], node_b + edge_b])
    coef2 = jnp.stack([edge_W[0], edge_b])
    zeros = jnp.zeros((RPT, H), jnp.float32)

    agg1 = sc_l1(pk, ewk, node_weight, coef1, zeros)
    x2 = _tc1(node_weight.reshape(N, 1), agg1,
              node_W, node_b.reshape(1, H),
              c1_W1, c1_b1.reshape(1, H), c1_g.reshape(1, H),
              c1_be.reshape(1, H), c1_W2, c1_b2.reshape(1, H))
    agg2 = sc_l2(pk, ewk, x2, coef2, zeros)
    out = _tc2(x2, agg2, batch.reshape(N, 1),
               c2_W1, c2_b1.reshape(1, H), c2_g.reshape(1, H),
               c2_be.reshape(1, H), c2_W2, c2_b2.reshape(1, H),
               m_W1, m_b1.reshape(1, H), m_g.reshape(1, H),
               m_be.reshape(1, H), m_W2, m_b2.reshape(1, OUT))
    return out


# speedup vs baseline: 1.3219x; 1.3219x over previous
import functools

import jax
import jax.numpy as jnp
from jax import lax
from jax.experimental import pallas as pl
from jax.experimental.pallas import tpu as pltpu
from jax.experimental.pallas import tpu_sc as plsc

H = 128
N = 10000
E = 320000
G = 64
OUT = 5

NC = 2
NS = 16
L = 16
NW = NC * NS
EPW = E // NW
CK = 80
NCHUNK = EPW // CK
NBUF = 4
NSUP = NCHUNK // NBUF
NTAIL = NCHUNK - NSUP * NBUF
SCAT_BYTES = CK * H * 4
RPT = 624
TAIL = N - NS * RPT

_NV = H // L

_GD = lax.GatherDimensionNumbers(
    offset_dims=(), collapsed_slice_dims=(0,), start_index_map=(0,))


def _bcast_lane(vec, j):
    idx = jnp.full((L, 1), j, jnp.int32)
    return lax.gather(vec, idx, _GD, (1,),
                      mode=lax.GatherScatterMode.PROMISE_IN_BOUNDS)


def _zero_acc(s, zero_h, acc):
    pltpu.sync_copy(zero_h.at[pl.ds(0, RPT)], acc.at[pl.ds(s * RPT, RPT)])

    @pl.when(s == 0)
    def _zero_tail():
        pltpu.sync_copy(zero_h.at[pl.ds(0, TAIL)], acc.at[pl.ds(NS * RPT, TAIL)])


def _write_out(c, s, acc, out_h):
    pltpu.sync_copy(acc.at[pl.ds(s * RPT, RPT)],
                    out_h.at[c, pl.ds(s * RPT, RPT)])

    @pl.when(s == 0)
    def _write_tail():
        pltpu.sync_copy(acc.at[pl.ds(NS * RPT, TAIL)],
                        out_h.at[c, pl.ds(NS * RPT, TAIL)])


def _sc_layer1(pk_h, ew_h, nw_h, coef_h, zero_h, out_h, *scr):
    nwg = scr[0:NBUF]
    mbuf = scr[NBUF:2 * NBUF]
    pidxA, pidxB, ewA, ewB, coefv, acc, semP, semG, semS = scr[2 * NBUF:]
    c = lax.axis_index("c")
    s = lax.axis_index("s")
    wid = c * NS + s
    pltpu.sync_copy(coef_h, coefv)
    _zero_acc(s, zero_h, acc)
    plsc.subcore_barrier()
    w0 = [coefv[0, pl.ds(L * v, L)] for v in range(_NV)]
    we = [coefv[1, pl.ds(L * v, L)] for v in range(_NV)]
    cc = [coefv[2, pl.ds(L * v, L)] for v in range(_NV)]

    def chunk_compute(b, ev):
        def group(q, cr):
            b16 = q * L
            nw16 = nwg[b][pl.ds(b16, L)]
            ew16 = ev[b, 0, pl.ds(b16, L)]

            def edge(jj, cj):
                j0 = jj * 2
                nwb0 = _bcast_lane(nw16, j0)
                ewb0 = _bcast_lane(ew16, j0)
                nwb1 = _bcast_lane(nw16, j0 + 1)
                ewb1 = _bcast_lane(ew16, j0 + 1)
                for v in range(_NV):
                    t0 = nwb0 * w0[v] + (ewb0 * we[v] + cc[v])
                    t1 = nwb1 * w0[v] + (ewb1 * we[v] + cc[v])
                    mbuf[b][b16 + j0, pl.ds(L * v, L)] = jnp.maximum(t0, 0.0)
                    mbuf[b][b16 + j0 + 1, pl.ds(L * v, L)] = jnp.maximum(t1, 0.0)
                return cj

            lax.fori_loop(0, L // 2, edge, 0)
            return cr

        lax.fori_loop(0, CK // L, group, 0)

    def gfire(p, b):
        return pltpu.async_copy(nw_h.at[p.at[b, 0]], nwg[b], semG.at[b])

    def gwait(p, b):
        pltpu.make_async_copy(nw_h.at[p.at[b, 0]], nwg[b], semG.at[b]).wait()

    def sfire(p, b):
        pltpu.async_copy(mbuf[b], acc.at[p.at[b, 1]], semS.at[b], add=True)

    def swait(p, b):
        pltpu.make_async_copy(mbuf[b], acc.at[p.at[b, 1]], semS.at[b]).wait()

    def lfire(gc, p, e, pf):
        pltpu.async_copy(pk_h.at[pl.ds(gc, NBUF)], p, semP.at[pf])
        pltpu.async_copy(ew_h.at[pl.ds(gc, NBUF)], e, semP.at[pf])

    def lwait(gc, p, e, pf):
        pltpu.make_async_copy(pk_h.at[pl.ds(gc, NBUF)], p, semP.at[pf]).wait()
        pltpu.make_async_copy(ew_h.at[pl.ds(gc, NBUF)], e, semP.at[pf]).wait()

    def phase(pcur, ecur, gc_next, pnxt, enxt, pf, prefire):
        for b in range(NBUF):
            gwait(pcur, b)
            chunk_compute(b, ecur)
            sfire(pcur, b)
            if prefire:
                if b == 0:
                    lfire(gc_next, pnxt, enxt, pf)
                if b == 1:
                    lwait(gc_next, pnxt, enxt, pf)
                if b >= 1:
                    swait(pcur, b - 1)
                    gfire(pnxt, b - 1)
        if prefire:
            swait(pcur, NBUF - 1)
            gfire(pnxt, NBUF - 1)
        else:
            for b in range(NBUF):
                swait(pcur, b)

    gc0 = wid * NCHUNK
    pltpu.sync_copy(pk_h.at[pl.ds(gc0, NBUF)], pidxA)
    pltpu.sync_copy(ew_h.at[pl.ds(gc0, NBUF)], ewA)
    for b in range(NBUF):
        gfire(pidxA, b)

    def body2(m, carry):
        ga = wid * NCHUNK + (2 * m) * NBUF
        phase(pidxA, ewA, ga + NBUF, pidxB, ewB, 1, True)
        phase(pidxB, ewB, ga + 2 * NBUF, pidxA, ewA, 0, True)
        return carry

    lax.fori_loop(0, NSUP // 2, body2, 0)
    phase(pidxA, ewA, 0, pidxB, ewB, 1, False)
    for t in range(NTAIL):
        gc = wid * NCHUNK + NSUP * NBUF + t
        pltpu.sync_copy(pk_h.at[pl.ds(gc, 1)], pidxA.at[pl.ds(0, 1)])
        pltpu.sync_copy(ew_h.at[pl.ds(gc, 1)], ewA.at[pl.ds(0, 1)])
        pltpu.async_copy(nw_h.at[pidxA.at[0, 0]], nwg[0], semG.at[0]).wait()
        chunk_compute(0, ewA)
        pltpu.sync_copy(mbuf[0], acc.at[pidxA.at[0, 1]], add=True)
    plsc.subcore_barrier()
    _write_out(c, s, acc, out_h)


def _sc_layer2(pk_h, ew_h, x_h, coef_h, zero_h, out_h, *scr):
    rows = scr[0:NBUF]
    pidxA, pidxB, ewA, ewB, coefv, acc, semP, semG, semS = scr[NBUF:]
    c = lax.axis_index("c")
    s = lax.axis_index("s")
    wid = c * NS + s
    pltpu.sync_copy(coef_h, coefv)
    _zero_acc(s, zero_h, acc)
    plsc.subcore_barrier()
    we = [coefv[0, pl.ds(L * v, L)] for v in range(_NV)]
    eb = [coefv[1, pl.ds(L * v, L)] for v in range(_NV)]

    def chunk_compute(b, ev):
        def group(q, cr):
            b16 = q * L
            ew16 = ev[b, 0, pl.ds(b16, L)]

            def edge(jj, cj):
                j0 = jj * 2
                ewb0 = _bcast_lane(ew16, j0)
                ewb1 = _bcast_lane(ew16, j0 + 1)
                for v in range(_NV):
                    sl = pl.ds(L * v, L)
                    t0 = rows[b][b16 + j0, sl] + (ewb0 * we[v] + eb[v])
                    t1 = rows[b][b16 + j0 + 1, sl] + (ewb1 * we[v] + eb[v])
                    rows[b][b16 + j0, sl] = jnp.maximum(t0, 0.0)
                    rows[b][b16 + j0 + 1, sl] = jnp.maximum(t1, 0.0)
                return cj

            lax.fori_loop(0, L // 2, edge, 0)
            return cr

        lax.fori_loop(0, CK // L, group, 0)

    def gfire(p, b):
        return pltpu.async_copy(x_h.at[p.at[b, 0]], rows[b], semG.at[b])

    def gwait(p, b):
        pltpu.make_async_copy(x_h.at[p.at[b, 0]], rows[b], semG.at[b]).wait()

    def sfire(p, b):
        pltpu.async_copy(rows[b], acc.at[p.at[b, 1]], semS.at[b], add=True)

    def swait(p, b):
        pltpu.make_async_copy(rows[b], acc.at[p.at[b, 1]], semS.at[b]).wait()

    def lfire(gc, p, e, pf):
        pltpu.async_copy(pk_h.at[pl.ds(gc, NBUF)], p, semP.at[pf])
        pltpu.async_copy(ew_h.at[pl.ds(gc, NBUF)], e, semP.at[pf])

    def lwait(gc, p, e, pf):
        pltpu.make_async_copy(pk_h.at[pl.ds(gc, NBUF)], p, semP.at[pf]).wait()
        pltpu.make_async_copy(ew_h.at[pl.ds(gc, NBUF)], e, semP.at[pf]).wait()

    def phase(pcur, ecur, gc_next, pnxt, enxt, pf, prefire):
        for b in range(NBUF):
            gwait(pcur, b)
            chunk_compute(b, ecur)
            sfire(pcur, b)
            if prefire:
                if b == 0:
                    lfire(gc_next, pnxt, enxt, pf)
                if b == 1:
                    lwait(gc_next, pnxt, enxt, pf)
                if b >= 1:
                    swait(pcur, b - 1)
                    gfire(pnxt, b - 1)
        if prefire:
            swait(pcur, NBUF - 1)
            gfire(pnxt, NBUF - 1)
        else:
            for b in range(NBUF):
                swait(pcur, b)

    gc0 = wid * NCHUNK
    pltpu.sync_copy(pk_h.at[pl.ds(gc0, NBUF)], pidxA)
    pltpu.sync_copy(ew_h.at[pl.ds(gc0, NBUF)], ewA)
    for b in range(NBUF):
        gfire(pidxA, b)

    def body2(m, carry):
        ga = wid * NCHUNK + (2 * m) * NBUF
        phase(pidxA, ewA, ga + NBUF, pidxB, ewB, 1, True)
        phase(pidxB, ewB, ga + 2 * NBUF, pidxA, ewA, 0, True)
        return carry

    lax.fori_loop(0, NSUP // 2, body2, 0)
    phase(pidxA, ewA, 0, pidxB, ewB, 1, False)
    for t in range(NTAIL):
        gc = wid * NCHUNK + NSUP * NBUF + t
        pltpu.sync_copy(pk_h.at[pl.ds(gc, 1)], pidxA.at[pl.ds(0, 1)])
        pltpu.sync_copy(ew_h.at[pl.ds(gc, 1)], ewA.at[pl.ds(0, 1)])
        pltpu.async_copy(x_h.at[pidxA.at[0, 0]], rows[0], semG.at[0]).wait()
        chunk_compute(0, ewA)
        pltpu.sync_copy(rows[0], acc.at[pidxA.at[0, 1]], add=True)
    plsc.subcore_barrier()
    _write_out(c, s, acc, out_h)


@functools.cache
def _build_sc_kernels():
    mesh = plsc.VectorSubcoreMesh(
        core_axis_name="c", subcore_axis_name="s",
        num_cores=NC, num_subcores=NS)
    def ring_scratch(n_row_rings, coef_rows):
        return (
            [pltpu.VMEM((CK,), jnp.float32)
             for _ in range((n_row_rings - 1) * NBUF)]
            + [pltpu.VMEM((CK, H), jnp.float32) for _ in range(NBUF)]
            + [
                pltpu.VMEM((NBUF, 2, CK), jnp.int32),
                pltpu.VMEM((NBUF, 2, CK), jnp.int32),
                pltpu.VMEM((NBUF, 1, CK), jnp.float32),
                pltpu.VMEM((NBUF, 1, CK), jnp.float32),
                pltpu.VMEM((coef_rows, H), jnp.float32),
                pltpu.VMEM_SHARED((N, H), jnp.float32),
                pltpu.SemaphoreType.DMA((2,)),
                pltpu.SemaphoreType.DMA((NBUF,)),
                pltpu.SemaphoreType.DMA((NBUF,)),
            ]
        )

    sc_l1 = pl.kernel(
        _sc_layer1,
        out_type=jax.ShapeDtypeStruct((NC, N, H), jnp.float32),
        mesh=mesh,
        scratch_types=ring_scratch(2, 3),
    )
    sc_l2 = pl.kernel(
        _sc_layer2,
        out_type=jax.ShapeDtypeStruct((NC, N, H), jnp.float32),
        mesh=mesh,
        scratch_types=ring_scratch(1, 2),
    )
    return sc_l1, sc_l2


def _bn_rows(t, g, be):
    m = jnp.mean(t, axis=0, keepdims=True)
    v = jnp.mean((t - m) ** 2, axis=0, keepdims=True)
    return g * (t - m) * lax.rsqrt(v + 1e-5) + be


def _tc_dense1(nw_ref, agg_ref, w0_ref, b0_ref, w1_ref, b1_ref,
               g1_ref, be1_ref, w2_ref, b2_ref, out_ref):
    x = nw_ref[...] * w0_ref[...] + b0_ref[...]
    h = x + agg_ref[0] + agg_ref[1]
    t = jnp.dot(h, w1_ref[...], preferred_element_type=jnp.float32) + b1_ref[...]
    t = jnp.maximum(t, 0.0)
    tn = _bn_rows(t, g1_ref[...], be1_ref[...])
    y = jnp.dot(tn, w2_ref[...], preferred_element_type=jnp.float32) + b2_ref[...]
    out_ref[...] = jnp.maximum(y, 0.0)


def _tc_dense2(x_ref, agg_ref, batch_ref, w1_ref, b1_ref, g1_ref, be1_ref,
               w2_ref, b2_ref, mw1_ref, mb1_ref, mg_ref, mbe_ref,
               mw2_ref, mb2_ref, out_ref):
    h = x_ref[...] + agg_ref[0] + agg_ref[1]
    t = jnp.dot(h, w1_ref[...], preferred_element_type=jnp.float32) + b1_ref[...]
    t = jnp.maximum(t, 0.0)
    tn = _bn_rows(t, g1_ref[...], be1_ref[...])
    y = jnp.dot(tn, w2_ref[...], preferred_element_type=jnp.float32) + b2_ref[...]
    x3 = jnp.maximum(y, 0.0)
    oh = (batch_ref[...] == lax.broadcasted_iota(jnp.int32, (1, G), 1))
    pooled = lax.dot_general(oh.astype(jnp.float32), x3,
                             (((0,), (0,)), ((), ())),
                             preferred_element_type=jnp.float32)
    hh = jnp.dot(pooled, mw1_ref[...], preferred_element_type=jnp.float32)
    hh = jnp.maximum(hh + mb1_ref[...], 0.0)
    hn = _bn_rows(hh, mg_ref[...], mbe_ref[...])
    logits = jnp.dot(hn, mw2_ref[...], preferred_element_type=jnp.float32)
    logits = logits + mb2_ref[...]
    z = logits - jnp.max(logits, axis=1, keepdims=True)
    out_ref[...] = z - jnp.log(jnp.sum(jnp.exp(z), axis=1, keepdims=True))


_tc1 = pl.pallas_call(
    _tc_dense1, out_shape=jax.ShapeDtypeStruct((N, H), jnp.float32))
_tc2 = pl.pallas_call(
    _tc_dense2, out_shape=jax.ShapeDtypeStruct((G, OUT), jnp.float32))


def kernel(edge_index, edge_weight, node_weight, batch,
           node_W, node_b, edge_W, edge_b,
           c1_W1, c1_b1, c1_g, c1_be, c1_W2, c1_b2,
           c2_W1, c2_b1, c2_g, c2_be, c2_W2, c2_b2,
           m_W1, m_b1, m_g, m_be, m_W2, m_b2):
    sc_l1, sc_l2 = _build_sc_kernels()
    src = edge_index[0]
    dst = edge_index[1]
    pk = jnp.stack([src.reshape(-1, CK), dst.reshape(-1, CK)], axis=1)
    ewk = edge_weight.reshape(-1, 1, CK)
    coef1 = jnp.stack([node_W[0], edge_W[0], node_b + edge_b])
    coef2 = jnp.stack([edge_W[0], edge_b])
    zeros = jnp.zeros((RPT, H), jnp.float32)

    agg1 = sc_l1(pk, ewk, node_weight, coef1, zeros)
    x2 = _tc1(node_weight.reshape(N, 1), agg1,
              node_W, node_b.reshape(1, H),
              c1_W1, c1_b1.reshape(1, H), c1_g.reshape(1, H),
              c1_be.reshape(1, H), c1_W2, c1_b2.reshape(1, H))
    agg2 = sc_l2(pk, ewk, x2, coef2, zeros)
    out = _tc2(x2, agg2, batch.reshape(N, 1),
               c2_W1, c2_b1.reshape(1, H), c2_g.reshape(1, H),
               c2_be.reshape(1, H), c2_W2, c2_b2.reshape(1, H),
               m_W1, m_b1.reshape(1, H), m_g.reshape(1, H),
               m_be.reshape(1, H), m_W2, m_b2.reshape(1, OUT))
    return out
